# baseline (speedup 1.0000x reference)
import functools
import jax
import jax.numpy as jnp
from jax import lax
from jax.experimental import pallas as pl
from jax.experimental.pallas import tpu as pltpu
from jax.experimental.pallas import tpu_sc as plsc

D = 128
B_ROWS, T_COLS = 4096, 200
B_TOTAL = B_ROWS * T_COLS
NC, NS = 2, 16
NW = NC * NS
B_PER_W = B_TOTAL // NW
CHUNK = 128
SROWS = 64
G = B_PER_W // CHUNK
NB = 2
PE_ROWS = 367
L = 16


@jax.jit
def _sc_gather(doy_r, pe):
    mesh = plsc.VectorSubcoreMesh(core_axis_name="c", subcore_axis_name="s")

    @functools.partial(
        pl.kernel,
        out_type=jax.ShapeDtypeStruct((B_TOTAL, D), jnp.float32),
        mesh=mesh,
        compiler_params=pltpu.CompilerParams(needs_layout_passes=False),
        scratch_types=[
            pltpu.VMEM((G, CHUNK), jnp.int32),
            pltpu.VMEM((NB, CHUNK, D), jnp.float32),
            pltpu.VMEM((PE_ROWS, D), jnp.float32),
            pltpu.VMEM_SHARED((PE_ROWS, D), jnp.float32),
            pltpu.SemaphoreType.DMA,
            pltpu.SemaphoreType.DMA,
        ],
    )
    def k(doy_hbm, pe_hbm, out_hbm, idx_v, rows_v, pe_vm, pe_spm, gsem, ssem):
        sid = lax.axis_index("s")
        wid = sid * NC + lax.axis_index("c")
        base = wid * B_PER_W

        @pl.when(sid == 0)
        def _():
            pltpu.sync_copy(pe_hbm, pe_spm)

        pltpu.sync_copy(pe_hbm, pe_vm)
        pltpu.sync_copy(doy_hbm.at[wid], idx_v)
        plsc.subcore_barrier()

        iota = lax.iota(jnp.int32, L)
        cols = [c * L + iota for c in range(D // L)]

        def full16(x):
            return jnp.full((L,), x, jnp.int32)

        def start_sgather(g, b):
            pltpu.async_copy(pe_spm.at[idx_v.at[g].at[pl.ds(0, SROWS)]],
                             rows_v.at[b].at[pl.ds(0, SROWS)], gsem)

        def wait_sgather(g, b):
            pltpu.make_async_copy(pe_spm.at[idx_v.at[g].at[pl.ds(0, SROWS)]],
                                  rows_v.at[b].at[pl.ds(0, SROWS)], gsem).wait()

        def vgather(g, b):
            for r in range(SROWS, CHUNK):
                rsplat = plsc.load_gather(idx_v, [full16(g), full16(r)])
                row_ref = rows_v.at[b].at[r]
                for c in range(D // L):
                    row_ref[pl.ds(c * L, L)] = plsc.load_gather(pe_vm, [rsplat, cols[c]])

        def start_scatter(g, b):
            pltpu.async_copy(rows_v.at[b],
                             out_hbm.at[pl.ds(base + g * CHUNK, CHUNK)], ssem)

        def wait_one_scatter():
            pltpu.make_async_copy(rows_v.at[0],
                                  out_hbm.at[pl.ds(base, CHUNK)], ssem).wait()

        def do_chunk(g, b, drain):
            if drain:
                wait_one_scatter()
            start_sgather(g, b)
            vgather(g, b)
            wait_sgather(g, b)
            start_scatter(g, b)

        for g in range(NB):
            do_chunk(g, g, drain=False)

        def body(o, _):
            for j in range(NB):
                g = NB + o * NB + j
                do_chunk(g, j, drain=True)
            return ()

        lax.fori_loop(0, (G - NB) // NB, body, (), unroll=False)

        for _ in range(NB):
            wait_one_scatter()

    return k(doy_r, pe)


def kernel(doy, pe):
    doy_r = doy.reshape(NW, G, CHUNK).astype(jnp.int32)
    out = _sc_gather(doy_r, pe)
    return out.reshape(B_ROWS, T_COLS, D)


# speedup vs baseline: 2.5963x; 2.5963x over previous
import functools
import jax
import jax.numpy as jnp
from jax import lax
from jax.experimental import pallas as pl
from jax.experimental.pallas import tpu as pltpu
from jax.experimental.pallas import tpu_sc as plsc

D = 128
B_ROWS, T_COLS = 4096, 200
B_TOTAL = B_ROWS * T_COLS
NC, NS = 2, 16
NW = NC * NS
B_PER_W = B_TOTAL // NW
CHUNK = 128
GPC = 1
OUT_CHUNK = CHUNK * GPC
N_IDX = B_PER_W // CHUNK
G = B_PER_W // OUT_CHUNK
NB = 6
PE_ROWS = 367


@jax.jit
def _sc_gather(doy_r, pe):
    mesh = plsc.VectorSubcoreMesh(core_axis_name="c", subcore_axis_name="s")

    @functools.partial(
        pl.kernel,
        out_type=jax.ShapeDtypeStruct((B_TOTAL, D), jnp.float32),
        mesh=mesh,
        scratch_types=[
            pltpu.VMEM((N_IDX, CHUNK), jnp.int32),
            pltpu.VMEM((NB, OUT_CHUNK, D), jnp.float32),
            pltpu.VMEM_SHARED((PE_ROWS, D), jnp.float32),
            pltpu.SemaphoreType.DMA,
            pltpu.SemaphoreType.DMA,
        ],
    )
    def k(doy_hbm, pe_hbm, out_hbm, idx_v, rows_v, pe_spm, gsem, ssem):
        sid = lax.axis_index("s")
        wid = sid * NC + lax.axis_index("c")
        base = wid * B_PER_W

        @pl.when(sid == 0)
        def _():
            pltpu.sync_copy(pe_hbm, pe_spm)

        pltpu.sync_copy(doy_hbm.at[wid], idx_v)
        plsc.subcore_barrier()

        def start_gathers(g, b):
            for j in range(GPC):
                pltpu.async_copy(pe_spm.at[idx_v.at[GPC * g + j]],
                                 rows_v.at[b].at[pl.ds(j * CHUNK, CHUNK)], gsem)

        def wait_gathers(g, b):
            for j in range(GPC):
                pltpu.make_async_copy(pe_spm.at[idx_v.at[GPC * g + j]],
                                      rows_v.at[b].at[pl.ds(j * CHUNK, CHUNK)], gsem).wait()

        def start_scatter(g, b):
            pltpu.async_copy(rows_v.at[b],
                             out_hbm.at[pl.ds(base + g * OUT_CHUNK, OUT_CHUNK)], ssem)

        def wait_one_scatter():
            pltpu.make_async_copy(rows_v.at[0],
                                  out_hbm.at[pl.ds(base, OUT_CHUNK)], ssem).wait()

        for b in range(NB):
            start_gathers(b, b)
        for g in range(NB - 1):
            wait_gathers(g, g)
            start_scatter(g, g)

        def body(o, _):
            for j in range(NB):
                g = (NB - 1) + o * NB + j
                buf = (NB - 1 + j) % NB
                wait_gathers(g, buf)
                start_scatter(g, buf)
                wait_one_scatter()
                start_gathers(g + 1, (buf + 1) % NB)
            return ()

        n_main = (G - NB) // NB * NB
        lax.fori_loop(0, n_main // NB, body, (), unroll=False)

        for g in range(NB - 1 + n_main, G - 1):
            wait_gathers(g, g % NB)
            start_scatter(g, g % NB)
            wait_one_scatter()
            start_gathers(g + 1, (g + 1) % NB)

        wait_gathers(G - 1, (G - 1) % NB)
        start_scatter(G - 1, (G - 1) % NB)
        for _ in range(NB):
            wait_one_scatter()

    return k(doy_r, pe)


def kernel(doy, pe):
    doy_r = doy.reshape(NW, N_IDX, CHUNK).astype(jnp.int32)
    out = _sc_gather(doy_r, pe)
    return out.reshape(B_ROWS, T_COLS, D)


# speedup vs baseline: 2.7048x; 1.0418x over previous
import functools
import jax
import jax.numpy as jnp
from jax import lax
from jax.experimental import pallas as pl
from jax.experimental.pallas import tpu as pltpu
from jax.experimental.pallas import tpu_sc as plsc

D = 128
B_ROWS, T_COLS = 4096, 200
B_TOTAL = B_ROWS * T_COLS
NC, NS = 2, 16
NW = NC * NS
B_PER_W = B_TOTAL // NW
CHUNK = 128
GPC = 2
OUT_CHUNK = CHUNK * GPC
N_IDX = B_PER_W // CHUNK
G = B_PER_W // OUT_CHUNK
NB = 3
PE_ROWS = 367


@jax.jit
def _sc_gather(doy_r, pe):
    mesh = plsc.VectorSubcoreMesh(core_axis_name="c", subcore_axis_name="s")

    @functools.partial(
        pl.kernel,
        out_type=jax.ShapeDtypeStruct((B_TOTAL, D), jnp.float32),
        mesh=mesh,
        scratch_types=[
            pltpu.VMEM((N_IDX, CHUNK), jnp.int32),
            pltpu.VMEM((NB, OUT_CHUNK, D), jnp.float32),
            pltpu.VMEM_SHARED((PE_ROWS, D), jnp.float32),
            pltpu.SemaphoreType.DMA,
            pltpu.SemaphoreType.DMA,
        ],
    )
    def k(doy_hbm, pe_hbm, out_hbm, idx_v, rows_v, pe_spm, gsem, ssem):
        sid = lax.axis_index("s")
        wid = sid * NC + lax.axis_index("c")
        base = wid * B_PER_W

        @pl.when(sid == 0)
        def _():
            pltpu.sync_copy(pe_hbm, pe_spm)

        pltpu.sync_copy(doy_hbm.at[wid], idx_v)
        plsc.subcore_barrier()

        def start_gathers(g, b):
            for j in range(GPC):
                pltpu.async_copy(pe_spm.at[idx_v.at[GPC * g + j]],
                                 rows_v.at[b].at[pl.ds(j * CHUNK, CHUNK)], gsem)

        def wait_gathers(g, b):
            for j in range(GPC):
                pltpu.make_async_copy(pe_spm.at[idx_v.at[GPC * g + j]],
                                      rows_v.at[b].at[pl.ds(j * CHUNK, CHUNK)], gsem).wait()

        def start_scatter(g, b):
            pltpu.async_copy(rows_v.at[b],
                             out_hbm.at[pl.ds(base + g * OUT_CHUNK, OUT_CHUNK)], ssem)

        def wait_one_scatter():
            pltpu.make_async_copy(rows_v.at[0],
                                  out_hbm.at[pl.ds(base, OUT_CHUNK)], ssem).wait()

        for b in range(NB):
            start_gathers(b, b)
        for g in range(NB - 1):
            wait_gathers(g, g)
            start_scatter(g, g)

        def body(o, _):
            for j in range(NB):
                g = (NB - 1) + o * NB + j
                buf = (NB - 1 + j) % NB
                wait_gathers(g, buf)
                start_scatter(g, buf)
                wait_one_scatter()
                start_gathers(g + 1, (buf + 1) % NB)
            return ()

        n_main = (G - NB) // NB * NB
        lax.fori_loop(0, n_main // NB, body, (), unroll=False)

        for g in range(NB - 1 + n_main, G - 1):
            wait_gathers(g, g % NB)
            start_scatter(g, g % NB)
            wait_one_scatter()
            start_gathers(g + 1, (g + 1) % NB)

        wait_gathers(G - 1, (G - 1) % NB)
        start_scatter(G - 1, (G - 1) % NB)
        for _ in range(NB):
            wait_one_scatter()

    return k(doy_r, pe)


def kernel(doy, pe):
    doy_r = doy.reshape(NW, N_IDX, CHUNK).astype(jnp.int32)
    out = _sc_gather(doy_r, pe)
    return out.reshape(B_ROWS, T_COLS, D)


# speedup vs baseline: 2.7423x; 1.0138x over previous
import functools
import jax
import jax.numpy as jnp
from jax import lax
from jax.experimental import pallas as pl
from jax.experimental.pallas import tpu as pltpu
from jax.experimental.pallas import tpu_sc as plsc

D = 128
B_ROWS, T_COLS = 4096, 200
B_TOTAL = B_ROWS * T_COLS
NC, NS = 2, 16
NW = NC * NS
R_PER_W = B_ROWS // NW
B_PER_W = R_PER_W * T_COLS
G = R_PER_W
SPLITS = ((0, 128), (128, 72))
NB = 3
PE_ROWS = 367


@jax.jit
def _sc_gather(doy, pe):
    mesh = plsc.VectorSubcoreMesh(core_axis_name="c", subcore_axis_name="s")

    @functools.partial(
        pl.kernel,
        out_type=jax.ShapeDtypeStruct((B_TOTAL, D), jnp.float32),
        mesh=mesh,
        scratch_types=[
            pltpu.VMEM((R_PER_W, T_COLS), jnp.int32),
            pltpu.VMEM((NB, T_COLS, D), jnp.float32),
            pltpu.VMEM_SHARED((PE_ROWS, D), jnp.float32),
            pltpu.SemaphoreType.DMA,
            pltpu.SemaphoreType.DMA,
        ],
    )
    def k(doy_hbm, pe_hbm, out_hbm, idx_v, rows_v, pe_spm, gsem, ssem):
        sid = lax.axis_index("s")
        wid = sid * NC + lax.axis_index("c")
        base = wid * B_PER_W

        @pl.when(sid == 0)
        def _():
            pltpu.sync_copy(pe_hbm, pe_spm)

        pltpu.sync_copy(doy_hbm.at[pl.ds(wid * R_PER_W, R_PER_W)], idx_v)
        plsc.subcore_barrier()

        def start_gathers(g, b):
            for off, n in SPLITS:
                pltpu.async_copy(pe_spm.at[idx_v.at[g].at[pl.ds(off, n)]],
                                 rows_v.at[b].at[pl.ds(off, n)], gsem)

        def wait_gathers(g, b):
            for off, n in SPLITS:
                pltpu.make_async_copy(pe_spm.at[idx_v.at[g].at[pl.ds(off, n)]],
                                      rows_v.at[b].at[pl.ds(off, n)], gsem).wait()

        def start_scatter(g, b):
            pltpu.async_copy(rows_v.at[b],
                             out_hbm.at[pl.ds(base + g * T_COLS, T_COLS)], ssem)

        def wait_one_scatter():
            pltpu.make_async_copy(rows_v.at[0],
                                  out_hbm.at[pl.ds(base, T_COLS)], ssem).wait()

        for b in range(NB):
            start_gathers(b, b)
        for g in range(NB - 1):
            wait_gathers(g, g)
            start_scatter(g, g)

        def body(o, _):
            for j in range(NB):
                g = (NB - 1) + o * NB + j
                buf = (NB - 1 + j) % NB
                wait_gathers(g, buf)
                start_scatter(g, buf)
                wait_one_scatter()
                start_gathers(g + 1, (buf + 1) % NB)
            return ()

        n_main = (G - NB) // NB * NB
        lax.fori_loop(0, n_main // NB, body, (), unroll=False)

        for g in range(NB - 1 + n_main, G - 1):
            wait_gathers(g, g % NB)
            start_scatter(g, g % NB)
            wait_one_scatter()
            start_gathers(g + 1, (g + 1) % NB)

        wait_gathers(G - 1, (G - 1) % NB)
        start_scatter(G - 1, (G - 1) % NB)
        for _ in range(NB):
            wait_one_scatter()

    return k(doy, pe)


def kernel(doy, pe):
    out = _sc_gather(doy.astype(jnp.int32), pe)
    return out.reshape(B_ROWS, T_COLS, D)


# speedup vs baseline: 2.7478x; 1.0020x over previous
import functools
import jax
import jax.numpy as jnp
from jax import lax
from jax.experimental import pallas as pl
from jax.experimental.pallas import tpu as pltpu
from jax.experimental.pallas import tpu_sc as plsc

D = 128
B_ROWS, T_COLS = 4096, 200
B_TOTAL = B_ROWS * T_COLS
NC, NS = 2, 16
NW = NC * NS
R_PER_W = B_ROWS // NW
B_PER_W = R_PER_W * T_COLS
G = R_PER_W
SPLITS = ((0, 128), (128, 72))
NB = 3
PE_ROWS = 367


@jax.jit
def _sc_gather(doy, pe):
    mesh = plsc.VectorSubcoreMesh(core_axis_name="c", subcore_axis_name="s")

    @functools.partial(
        pl.kernel,
        out_type=jax.ShapeDtypeStruct((B_TOTAL, D), jnp.float32),
        mesh=mesh,
        compiler_params=pltpu.CompilerParams(use_tc_tiling_on_sc=True),
        scratch_types=[
            pltpu.VMEM((R_PER_W, T_COLS), jnp.int32),
            pltpu.VMEM((NB, T_COLS, D), jnp.float32),
            pltpu.VMEM_SHARED((PE_ROWS, D), jnp.float32),
            pltpu.SemaphoreType.DMA,
            pltpu.SemaphoreType.DMA,
        ],
    )
    def k(doy_hbm, pe_hbm, out_hbm, idx_v, rows_v, pe_spm, gsem, ssem):
        sid = lax.axis_index("s")
        wid = sid * NC + lax.axis_index("c")
        base = wid * B_PER_W

        @pl.when(sid == 0)
        def _():
            pltpu.sync_copy(pe_hbm, pe_spm)

        pltpu.sync_copy(doy_hbm.at[pl.ds(wid * R_PER_W, R_PER_W)], idx_v)
        plsc.subcore_barrier()

        def start_gathers(g, b):
            for off, n in SPLITS:
                pltpu.async_copy(pe_spm.at[idx_v.at[g].at[pl.ds(off, n)]],
                                 rows_v.at[b].at[pl.ds(off, n)], gsem)

        def wait_gathers(g, b):
            for off, n in SPLITS:
                pltpu.make_async_copy(pe_spm.at[idx_v.at[g].at[pl.ds(off, n)]],
                                      rows_v.at[b].at[pl.ds(off, n)], gsem).wait()

        def start_scatter(g, b):
            pltpu.async_copy(rows_v.at[b],
                             out_hbm.at[pl.ds(base + g * T_COLS, T_COLS)], ssem)

        def wait_one_scatter():
            pltpu.make_async_copy(rows_v.at[0],
                                  out_hbm.at[pl.ds(base, T_COLS)], ssem).wait()

        for b in range(NB):
            start_gathers(b, b)
        for g in range(NB - 1):
            wait_gathers(g, g)
            start_scatter(g, g)

        def body(o, _):
            for j in range(NB):
                g = (NB - 1) + o * NB + j
                buf = (NB - 1 + j) % NB
                wait_gathers(g, buf)
                start_scatter(g, buf)
                wait_one_scatter()
                start_gathers(g + 1, (buf + 1) % NB)
            return ()

        n_main = (G - NB) // NB * NB
        lax.fori_loop(0, n_main // NB, body, (), unroll=False)

        for g in range(NB - 1 + n_main, G - 1):
            wait_gathers(g, g % NB)
            start_scatter(g, g % NB)
            wait_one_scatter()
            start_gathers(g + 1, (g + 1) % NB)

        wait_gathers(G - 1, (G - 1) % NB)
        start_scatter(G - 1, (G - 1) % NB)
        for _ in range(NB):
            wait_one_scatter()

    return k(doy, pe)


def kernel(doy, pe):
    out = _sc_gather(doy.astype(jnp.int32), pe)
    return out.reshape(B_ROWS, T_COLS, D)
